# Initial kernel scaffold; baseline (speedup 1.0000x reference)
import jax
import jax.numpy as jnp
from jax.experimental import pallas as pl


def kernel(positions, cos_sin_cache):
    raise NotImplementedError("write your pallas kernel here")


# speedup vs baseline: 1.9300x; 1.9300x over previous
import functools

import jax
import jax.numpy as jnp
from jax import lax
from jax.experimental import pallas as pl
from jax.experimental.pallas import tpu as pltpu
from jax.experimental.pallas import tpu_sc as plsc

HEAD_SIZE = 128
HALF = HEAD_SIZE // 2
BATCH = 4
SEQ = 8192
N = BATCH * SEQ
CHUNK = 128
N_ROWS = N // CHUNK


@functools.cache
def _build_sc_kernel():
    info = plsc.get_sparse_core_info()
    nc, ns = info.num_cores, info.num_subcores
    nw = nc * ns
    rows_w = N_ROWS // nw

    mesh = plsc.VectorSubcoreMesh(core_axis_name="c", subcore_axis_name="s")

    @functools.partial(
        pl.kernel,
        mesh=mesh,
        compiler_params=pltpu.CompilerParams(use_tc_tiling_on_sc=False),
        out_type=(
            jax.ShapeDtypeStruct((N, HALF), jnp.float32),
            jax.ShapeDtypeStruct((N, HALF), jnp.float32),
        ),
        scratch_types=[
            pltpu.VMEM((rows_w, CHUNK), jnp.int32),
            pltpu.VMEM((rows_w, CHUNK), jnp.int32),
            pltpu.VMEM((rows_w, CHUNK), jnp.int32),
            pltpu.VMEM((CHUNK, HALF), jnp.float32),
            pltpu.VMEM((CHUNK, HALF), jnp.float32),
            pltpu.SemaphoreType.DMA,
            pltpu.SemaphoreType.DMA,
        ],
    )
    def rotary_gather(pos_hbm, table_hbm, cos_hbm, sin_hbm,
                      pos_v, cidx_v, sidx_v, crows, srows, sem_c, sem_s):
        wid = lax.axis_index("s") * nc + lax.axis_index("c")
        row0 = wid * rows_w
        pltpu.sync_copy(pos_hbm.at[pl.ds(row0, rows_w)], pos_v)
        for j in range(rows_w):
            for i in range(CHUNK // 16):
                p = pos_v[j, pl.ds(i * 16, 16)]
                two_p = p + p
                cidx_v[j, pl.ds(i * 16, 16)] = two_p
                sidx_v[j, pl.ds(i * 16, 16)] = two_p + 1
        for j in range(rows_w):
            gc = pltpu.async_copy(table_hbm.at[cidx_v.at[j]], crows, sem_c)
            gs = pltpu.async_copy(table_hbm.at[sidx_v.at[j]], srows, sem_s)
            gc.wait()
            gs.wait()
            base = (row0 + j) * CHUNK
            pltpu.sync_copy(crows, cos_hbm.at[pl.ds(base, CHUNK)])
            pltpu.sync_copy(srows, sin_hbm.at[pl.ds(base, CHUNK)])

    return rotary_gather


def kernel(positions, cos_sin_cache):
    pos = positions.astype(jnp.int32).reshape(N_ROWS, CHUNK)
    table = cos_sin_cache.reshape(2 * 8192, HALF)
    cos_flat, sin_flat = _build_sc_kernel()(pos, table)
    cos = cos_flat.reshape(BATCH, 1, SEQ, HALF)
    sin = sin_flat.reshape(BATCH, 1, SEQ, HALF)
    return (cos, sin)


# speedup vs baseline: 2.0534x; 1.0640x over previous
import functools

import jax
import jax.numpy as jnp
from jax import lax
from jax.experimental import pallas as pl
from jax.experimental.pallas import tpu as pltpu
from jax.experimental.pallas import tpu_sc as plsc

HEAD_SIZE = 128
HALF = HEAD_SIZE // 2
BATCH = 4
SEQ = 8192
N = BATCH * SEQ
CHUNK = 128
N_ROWS = N // CHUNK
GROUP = 2
RING = 2


@functools.cache
def _build_sc_kernel():
    info = plsc.get_sparse_core_info()
    nc, ns = info.num_cores, info.num_subcores
    nw = nc * ns
    rows_w = N_ROWS // nw

    mesh = plsc.VectorSubcoreMesh(core_axis_name="c", subcore_axis_name="s")

    @functools.partial(
        pl.kernel,
        mesh=mesh,
        compiler_params=pltpu.CompilerParams(use_tc_tiling_on_sc=False),
        out_type=(
            jax.ShapeDtypeStruct((N, HALF), jnp.float32),
            jax.ShapeDtypeStruct((N, HALF), jnp.float32),
        ),
        scratch_types=[
            pltpu.VMEM((rows_w, CHUNK), jnp.int32),
            pltpu.VMEM((rows_w, CHUNK), jnp.int32),
            pltpu.VMEM((rows_w, CHUNK), jnp.int32),
            pltpu.VMEM((RING, GROUP * CHUNK, HALF), jnp.float32),
            pltpu.VMEM((RING, GROUP * CHUNK, HALF), jnp.float32),
            pltpu.SemaphoreType.DMA,
            pltpu.SemaphoreType.DMA,
            pltpu.SemaphoreType.DMA,
            pltpu.SemaphoreType.DMA,
        ],
    )
    def rotary_gather(pos_hbm, table_hbm, cos_hbm, sin_hbm,
                      pos_v, cidx_v, sidx_v, crows, srows,
                      sem_gc, sem_gs, sem_wc, sem_ws):
        wid = lax.axis_index("s") * nc + lax.axis_index("c")
        row0 = wid * rows_w
        n_groups = rows_w // GROUP
        pltpu.sync_copy(pos_hbm.at[pl.ds(row0, rows_w)], pos_v)
        for j in range(rows_w):
            for i in range(CHUNK // 16):
                p = pos_v[j, pl.ds(i * 16, 16)]
                two_p = p + p
                cidx_v[j, pl.ds(i * 16, 16)] = two_p
                sidx_v[j, pl.ds(i * 16, 16)] = two_p + 1

        def issue_gathers(g):
            b = g % RING
            ds = []
            for t in range(GROUP):
                j = g * GROUP + t
                dst = pl.ds(t * CHUNK, CHUNK)
                ds.append(pltpu.async_copy(
                    table_hbm.at[cidx_v.at[j]], crows.at[b, dst], sem_gc))
                ds.append(pltpu.async_copy(
                    table_hbm.at[sidx_v.at[j]], srows.at[b, dst], sem_gs))
            return ds

        def issue_writes(g):
            b = g % RING
            base = (row0 + g * GROUP) * CHUNK
            dst = pl.ds(base, GROUP * CHUNK)
            return (
                pltpu.async_copy(crows.at[b], cos_hbm.at[dst], sem_wc),
                pltpu.async_copy(srows.at[b], sin_hbm.at[dst], sem_ws),
            )

        g_desc = {0: issue_gathers(0)}
        w_desc = {}
        for g in range(n_groups):
            if g + 1 < n_groups:
                if g - 1 >= 0:
                    for d in w_desc[g - 1]:
                        d.wait()
                g_desc[g + 1] = issue_gathers(g + 1)
            for d in g_desc[g]:
                d.wait()
            w_desc[g] = issue_writes(g)
        for d in w_desc[n_groups - 2]:
            d.wait()
        for d in w_desc[n_groups - 1]:
            d.wait()

    return rotary_gather


def kernel(positions, cos_sin_cache):
    pos = positions.astype(jnp.int32).reshape(N_ROWS, CHUNK)
    table = cos_sin_cache.reshape(2 * 8192, HALF)
    cos_flat, sin_flat = _build_sc_kernel()(pos, table)
    cos = cos_flat.reshape(BATCH, 1, SEQ, HALF)
    sin = sin_flat.reshape(BATCH, 1, SEQ, HALF)
    return (cos, sin)
